# baseline (speedup 1.0000x reference)
import functools

import jax
import jax.numpy as jnp
from jax.experimental import pallas as pl
from jax.experimental.pallas import tpu as pltpu

N = 10000
D = 64
BM = 200


def _matmul_bias_kernel(a_ref, x_ref, b_ref, o_ref, *, prod_scale, bias_scale):
    o_ref[...] = prod_scale * jnp.dot(
        a_ref[...], x_ref[...], preferred_element_type=jnp.float32
    ) + bias_scale * b_ref[...]


def _layer(a, x, bias, prod_scale, bias_scale):
    return pl.pallas_call(
        functools.partial(_matmul_bias_kernel, prod_scale=prod_scale,
                          bias_scale=bias_scale),
        grid=(N // BM,),
        in_specs=[
            pl.BlockSpec((BM, N), lambda i: (i, 0)),
            pl.BlockSpec((N, D), lambda i: (0, 0)),
            pl.BlockSpec((BM, D), lambda i: (i, 0)),
        ],
        out_specs=pl.BlockSpec((BM, D), lambda i: (i, 0)),
        out_shape=jax.ShapeDtypeStruct((N, D), jnp.float32),
        compiler_params=pltpu.CompilerParams(
            dimension_semantics=("parallel",)),
    )(a, x, bias)


def kernel(user_embedding, item_embedding, norm_adj):
    alpha = 0.25
    x = jnp.concatenate([user_embedding, item_embedding], axis=0)
    r = _layer(norm_adj, x, x, alpha, alpha)
    r = _layer(norm_adj, r, x, 1.0, alpha)
    r = _layer(norm_adj, r, x, 1.0, alpha)
    return (r[:6000], r[6000:])
